# baseline (speedup 1.0000x reference)
import functools

import jax
import jax.numpy as jnp
from jax import lax
from jax.experimental import pallas as pl
from jax.experimental.pallas import tpu as pltpu
from jax.experimental.pallas import tpu_sc as plsc

N_MID = 1000000
USER_COUNT = 100000
DIM = 32
B = 4096
SEQ = 200

NC = 2
NS = 16
NW = NC * NS

BW = B // NW
NPAIR = SEQ // 2
L = 16

_mesh = plsc.VectorSubcoreMesh(
    core_axis_name="c", subcore_axis_name="s", num_cores=NC, num_subcores=NS
)


@functools.partial(
    pl.kernel,
    out_type=(
        jax.ShapeDtypeStruct((SEQ, DIM // 8, B // BW, 8, BW), jnp.float32),
        jax.ShapeDtypeStruct((DIM // 8, B // BW, 8, BW), jnp.float32),
    ),
    mesh=_mesh,
    scratch_types=[
        pltpu.VMEM((SEQ, BW), jnp.int32),
        pltpu.VMEM((BW, DIM), jnp.float32),
        pltpu.VMEM((BW, DIM), jnp.float32),
        pltpu.VMEM((DIM // 8, 8, BW), jnp.float32),
        pltpu.VMEM((DIM // 8, 8, BW), jnp.float32),
        pltpu.VMEM((BW,), jnp.int32),
        pltpu.VMEM((BW, DIM), jnp.float32),
        pltpu.SemaphoreType.DMA,
        pltpu.SemaphoreType.DMA,
        pltpu.SemaphoreType.DMA,
        pltpu.SemaphoreType.DMA,
        pltpu.SemaphoreType.DMA,
    ],
    compiler_params=pltpu.CompilerParams(
        use_tc_tiling_on_sc=False, needs_layout_passes=False
    ),
)
def _sc_gather(
    itemT_hbm, user_hbm, table_hbm, utable_hbm,
    item_out, user_out,
    idx_all, rowsa, rowsb, uoa, uob,
    uidx_v, urows_v,
    g0, g1, w0, w1, usem,
):
    wid = lax.axis_index("s") * NC + lax.axis_index("c")
    bbase = wid * BW
    rows = (rowsa, rowsb)
    uo = (uoa, uob)
    gsem = (g0, g1)
    wsem = (w0, w1)
    iota = lax.iota(jnp.int32, L)

    pltpu.sync_copy(itemT_hbm.at[:, pl.ds(bbase, BW)], idx_all)

    pltpu.sync_copy(user_hbm.at[pl.ds(bbase, BW)], uidx_v)
    ucopy = pltpu.async_copy(utable_hbm.at[uidx_v], urows_v, usem)

    def fire(s, b):
        pltpu.async_copy(table_hbm.at[idx_all.at[s]], rows[b], gsem[b])

    def drain_gather(b):
        pltpu.make_async_copy(table_hbm.at[pl.ds(0, BW)], rows[b], gsem[b]).wait()

    def wait_wb(b):
        pltpu.make_async_copy(item_out.at[0, :, 0], uo[b], wsem[b]).wait()

    def transpose_tile(src, dst):
        for k in range(BW // L):
            rvec = iota + (k * L)
            for c in range(DIM):
                cvec = jnp.full((L,), c, jnp.int32)
                v = plsc.load_gather(src, [rvec, cvec])
                dst[c // 8, c % 8, pl.ds(k * L, L)] = v

    def writeback(s, b):
        pltpu.async_copy(uo[b], item_out.at[s, :, wid], wsem[b])

    fire(0, 0)
    fire(1, 1)
    drain_gather(0)
    transpose_tile(rows[0], uo[0])
    writeback(0, 0)
    fire(2, 0)
    drain_gather(1)
    transpose_tile(rows[1], uo[1])
    writeback(1, 1)
    fire(3, 1)

    @pl.loop(1, NPAIR - 1)
    def _pair(i):
        s0 = 2 * i
        drain_gather(0)
        wait_wb(0)
        transpose_tile(rows[0], uo[0])
        writeback(s0, 0)
        fire(s0 + 2, 0)
        drain_gather(1)
        wait_wb(1)
        transpose_tile(rows[1], uo[1])
        writeback(s0 + 1, 1)
        fire(s0 + 3, 1)

    drain_gather(0)
    wait_wb(0)
    transpose_tile(rows[0], uo[0])
    writeback(SEQ - 2, 0)
    drain_gather(1)
    wait_wb(1)
    transpose_tile(rows[1], uo[1])
    writeback(SEQ - 1, 1)

    ucopy.wait()
    wait_wb(0)
    transpose_tile(urows_v, uoa)
    pltpu.async_copy(uoa, user_out.at[:, wid], w0)
    wait_wb(1)
    wait_wb(0)


def _mask_body(mask_ref, out_ref):
    out_ref[...] = jnp.sum(mask_ref[...], axis=0).astype(jnp.int32)


def kernel(item, nbr_mask, user_id, item_input_lookup, user_embedding_matrix):
    itemT = item.T
    o5, uo4 = _sc_gather(itemT, user_id, item_input_lookup, user_embedding_matrix)
    item_emb = jnp.transpose(o5, (2, 4, 0, 1, 3)).reshape(B, SEQ, DIM)
    user_embedding = jnp.transpose(uo4, (1, 3, 0, 2)).reshape(B, DIM)
    mask_length = pl.pallas_call(
        _mask_body,
        out_shape=jax.ShapeDtypeStruct((B,), jnp.int32),
    )(nbr_mask.T)
    return item_emb, user_embedding, mask_length


# speedup vs baseline: 1.0621x; 1.0621x over previous
import functools

import jax
import jax.numpy as jnp
from jax import lax
from jax.experimental import pallas as pl
from jax.experimental.pallas import tpu as pltpu
from jax.experimental.pallas import tpu_sc as plsc

N_MID = 1000000
USER_COUNT = 100000
DIM = 32
B = 4096
SEQ = 200

NC = 2
NS = 16
NW = NC * NS

BW = B // NW
NBUF = 8
NBLK = SEQ // NBUF
L = 16

_mesh = plsc.VectorSubcoreMesh(
    core_axis_name="c", subcore_axis_name="s", num_cores=NC, num_subcores=NS
)


@functools.partial(
    pl.kernel,
    out_type=(
        jax.ShapeDtypeStruct((SEQ, DIM // 8, B // BW, 8, BW), jnp.float32),
        jax.ShapeDtypeStruct((DIM // 8, B // BW, 8, BW), jnp.float32),
    ),
    mesh=_mesh,
    scratch_types=(
        [pltpu.VMEM((SEQ, BW), jnp.int32)]
        + [pltpu.VMEM((BW, DIM), jnp.float32)] * NBUF
        + [pltpu.VMEM((DIM // 8, 8, BW), jnp.float32)] * NBUF
        + [
            pltpu.VMEM((BW,), jnp.int32),
            pltpu.VMEM((BW, DIM), jnp.float32),
        ]
        + [pltpu.SemaphoreType.DMA] * (2 * NBUF + 1)
    ),
    compiler_params=pltpu.CompilerParams(
        use_tc_tiling_on_sc=False, needs_layout_passes=False
    ),
)
def _sc_gather(
    itemT_hbm, user_hbm, table_hbm, utable_hbm,
    item_out, user_out,
    idx_all, *ring,
):
    rows = ring[:NBUF]
    uo = ring[NBUF:2 * NBUF]
    uidx_v, urows_v = ring[2 * NBUF:2 * NBUF + 2]
    gsem = ring[2 * NBUF + 2:3 * NBUF + 2]
    wsem = ring[3 * NBUF + 2:4 * NBUF + 2]
    usem = ring[4 * NBUF + 2]
    wid = lax.axis_index("s") * NC + lax.axis_index("c")
    bbase = wid * BW
    iota = lax.iota(jnp.int32, L)

    pltpu.sync_copy(itemT_hbm.at[:, pl.ds(bbase, BW)], idx_all)

    pltpu.sync_copy(user_hbm.at[pl.ds(bbase, BW)], uidx_v)
    ucopy = pltpu.async_copy(utable_hbm.at[uidx_v], urows_v, usem)

    def fire(s, b):
        pltpu.async_copy(table_hbm.at[idx_all.at[s]], rows[b], gsem[b])

    def drain_gather(b):
        pltpu.make_async_copy(table_hbm.at[pl.ds(0, BW)], rows[b], gsem[b]).wait()

    def wait_wb(b):
        pltpu.make_async_copy(item_out.at[0, :, 0], uo[b], wsem[b]).wait()

    def transpose_tile(src, dst):
        @pl.loop(0, BW // L)
        def _k(k):
            rvec = iota + k * L
            for c in range(DIM):
                cvec = jnp.full((L,), c, jnp.int32)
                v = plsc.load_gather(src, [rvec, cvec])
                dst[c // 8, c % 8, pl.ds(k * L, L)] = v

    def writeback(s, b):
        pltpu.async_copy(uo[b], item_out.at[s, :, wid], wsem[b])

    for b in range(NBUF):
        fire(b, b)
    for b in range(NBUF):
        drain_gather(b)
        transpose_tile(rows[b], uo[b])
        writeback(b, b)
        fire(NBUF + b, b)

    @pl.loop(1, NBLK - 1)
    def _round(i):
        s0 = i * NBUF
        for b in range(NBUF):
            drain_gather(b)
            wait_wb(b)
            transpose_tile(rows[b], uo[b])
            writeback(s0 + b, b)
            fire(s0 + NBUF + b, b)

    for b in range(NBUF):
        drain_gather(b)
        wait_wb(b)
        transpose_tile(rows[b], uo[b])
        writeback(SEQ - NBUF + b, b)

    ucopy.wait()
    wait_wb(0)
    transpose_tile(urows_v, uo[0])
    pltpu.async_copy(uo[0], user_out.at[:, wid], wsem[0])
    for b in range(NBUF):
        wait_wb(b)


def _mask_body(mask_ref, out_ref):
    out_ref[...] = jnp.sum(mask_ref[...], axis=0).astype(jnp.int32)


def kernel(item, nbr_mask, user_id, item_input_lookup, user_embedding_matrix):
    itemT = item.T
    o5, uo4 = _sc_gather(itemT, user_id, item_input_lookup, user_embedding_matrix)
    item_emb = jnp.transpose(o5, (2, 4, 0, 1, 3)).reshape(B, SEQ, DIM)
    user_embedding = jnp.transpose(uo4, (1, 3, 0, 2)).reshape(B, DIM)
    mask_length = pl.pallas_call(
        _mask_body,
        out_shape=jax.ShapeDtypeStruct((B,), jnp.int32),
    )(nbr_mask.T)
    return item_emb, user_embedding, mask_length


# speedup vs baseline: 1.2472x; 1.1743x over previous
import functools

import jax
import jax.numpy as jnp
from jax import lax
from jax.experimental import pallas as pl
from jax.experimental.pallas import tpu as pltpu
from jax.experimental.pallas import tpu_sc as plsc

N_MID = 1000000
USER_COUNT = 100000
DIM = 32
B = 4096
SEQ = 200

NC = 2
NS = 16
NW = NC * NS

BW = B // NW
NBUF = 8
NBLK = SEQ // NBUF
L = 16

_mesh = plsc.VectorSubcoreMesh(
    core_axis_name="c", subcore_axis_name="s", num_cores=NC, num_subcores=NS
)


@functools.partial(
    pl.kernel,
    out_type=(
        jax.ShapeDtypeStruct((SEQ, DIM // 8, B // BW, 8, BW), jnp.float32),
        jax.ShapeDtypeStruct((DIM // 8, B // BW, 8, BW), jnp.float32),
    ),
    mesh=_mesh,
    scratch_types=(
        [pltpu.VMEM((SEQ, BW), jnp.int32)]
        + [pltpu.VMEM((BW, DIM), jnp.float32)] * NBUF
        + [pltpu.VMEM((L, DIM + 1), jnp.float32)]
        + [pltpu.VMEM((DIM // 8, 8, BW), jnp.float32)] * NBUF
        + [
            pltpu.VMEM((BW,), jnp.int32),
            pltpu.VMEM((BW, DIM), jnp.float32),
        ]
        + [pltpu.SemaphoreType.DMA] * (2 * NBUF + 1)
    ),
    compiler_params=pltpu.CompilerParams(
        use_tc_tiling_on_sc=False, needs_layout_passes=False
    ),
)
def _sc_gather(
    itemT_hbm, user_hbm, table_hbm, utable_hbm,
    item_out, user_out,
    idx_all, *ring,
):
    rows = ring[:NBUF]
    pad = ring[NBUF]
    uo = ring[NBUF + 1:2 * NBUF + 1]
    uidx_v, urows_v = ring[2 * NBUF + 1:2 * NBUF + 3]
    gsem = ring[2 * NBUF + 3:3 * NBUF + 3]
    wsem = ring[3 * NBUF + 3:4 * NBUF + 3]
    usem = ring[4 * NBUF + 3]
    wid = lax.axis_index("s") * NC + lax.axis_index("c")
    bbase = wid * BW
    iota = lax.iota(jnp.int32, L)

    pltpu.sync_copy(itemT_hbm.at[:, pl.ds(bbase, BW)], idx_all)

    pltpu.sync_copy(user_hbm.at[pl.ds(bbase, BW)], uidx_v)
    ucopy = pltpu.async_copy(utable_hbm.at[uidx_v], urows_v, usem)

    def fire(s, b):
        pltpu.async_copy(table_hbm.at[idx_all.at[s]], rows[b], gsem[b])

    def drain_gather(b):
        pltpu.make_async_copy(table_hbm.at[pl.ds(0, BW)], rows[b], gsem[b]).wait()

    def wait_wb(b):
        pltpu.make_async_copy(item_out.at[0, :, 0], uo[b], wsem[b]).wait()

    def transpose_tile(src, dst):
        @pl.loop(0, BW // L)
        def _k(k):
            for jj in range(L):
                pad[jj, pl.ds(0, L)] = src[k * L + jj, pl.ds(0, L)]
                pad[jj, pl.ds(L, L)] = src[k * L + jj, pl.ds(L, L)]
            for c in range(DIM):
                cvec = jnp.full((L,), c, jnp.int32)
                v = plsc.load_gather(pad, [iota, cvec])
                dst[c // 8, c % 8, pl.ds(k * L, L)] = v

    def writeback(s, b):
        pltpu.async_copy(uo[b], item_out.at[s, :, wid], wsem[b])

    for b in range(NBUF):
        fire(b, b)
    for b in range(NBUF):
        drain_gather(b)
        transpose_tile(rows[b], uo[b])
        writeback(b, b)
        fire(NBUF + b, b)

    @pl.loop(1, NBLK - 1)
    def _round(i):
        s0 = i * NBUF
        for b in range(NBUF):
            drain_gather(b)
            wait_wb(b)
            transpose_tile(rows[b], uo[b])
            writeback(s0 + b, b)
            fire(s0 + NBUF + b, b)

    for b in range(NBUF):
        drain_gather(b)
        wait_wb(b)
        transpose_tile(rows[b], uo[b])
        writeback(SEQ - NBUF + b, b)

    ucopy.wait()
    wait_wb(0)
    transpose_tile(urows_v, uo[0])
    pltpu.async_copy(uo[0], user_out.at[:, wid], wsem[0])
    for b in range(NBUF):
        wait_wb(b)


def _mask_body(mask_ref, out_ref):
    out_ref[...] = jnp.sum(mask_ref[...], axis=0).astype(jnp.int32)


def kernel(item, nbr_mask, user_id, item_input_lookup, user_embedding_matrix):
    itemT = item.T
    o5, uo4 = _sc_gather(itemT, user_id, item_input_lookup, user_embedding_matrix)
    item_emb = jnp.transpose(o5, (2, 4, 0, 1, 3)).reshape(B, SEQ, DIM)
    user_embedding = jnp.transpose(uo4, (1, 3, 0, 2)).reshape(B, DIM)
    mask_length = pl.pallas_call(
        _mask_body,
        out_shape=jax.ShapeDtypeStruct((B,), jnp.int32),
    )(nbr_mask.T)
    return item_emb, user_embedding, mask_length


# speedup vs baseline: 1.2587x; 1.0092x over previous
import functools

import jax
import jax.numpy as jnp
from jax import lax
from jax.experimental import pallas as pl
from jax.experimental.pallas import tpu as pltpu
from jax.experimental.pallas import tpu_sc as plsc

N_MID = 1000000
USER_COUNT = 100000
DIM = 32
B = 4096
SEQ = 200

NC = 2
NS = 16
NW = NC * NS

BW = B // NW
NBUF = 4
NBLK = SEQ // NBUF
L = 16

_mesh = plsc.VectorSubcoreMesh(
    core_axis_name="c", subcore_axis_name="s", num_cores=NC, num_subcores=NS
)


@functools.partial(
    pl.kernel,
    out_type=(
        jax.ShapeDtypeStruct((SEQ, DIM // 8, B // BW, 8, BW), jnp.float32),
        jax.ShapeDtypeStruct((DIM // 8, B // BW, 8, BW), jnp.float32),
    ),
    mesh=_mesh,
    scratch_types=(
        [pltpu.VMEM((SEQ, BW), jnp.int32)]
        + [pltpu.VMEM((BW, 128), jnp.float32)] * NBUF
        + [pltpu.VMEM((L, DIM + 1), jnp.float32)]
        + [pltpu.VMEM((DIM // 8, 8, BW), jnp.float32)] * NBUF
        + [
            pltpu.VMEM((BW,), jnp.int32),
            pltpu.VMEM((BW, 128), jnp.float32),
        ]
        + [pltpu.SemaphoreType.DMA] * (2 * NBUF + 1)
    ),
    compiler_params=pltpu.CompilerParams(
        use_tc_tiling_on_sc=False, needs_layout_passes=False
    ),
)
def _sc_gather(
    itemT_hbm, user_hbm, table_hbm, utable_hbm,
    item_out, user_out,
    idx_all, *ring,
):
    rows = ring[:NBUF]
    pad = ring[NBUF]
    uo = ring[NBUF + 1:2 * NBUF + 1]
    uidx_v, urows_v = ring[2 * NBUF + 1:2 * NBUF + 3]
    gsem = ring[2 * NBUF + 3:3 * NBUF + 3]
    wsem = ring[3 * NBUF + 3:4 * NBUF + 3]
    usem = ring[4 * NBUF + 3]
    wid = lax.axis_index("s") * NC + lax.axis_index("c")
    bbase = wid * BW
    iota = lax.iota(jnp.int32, L)

    pltpu.sync_copy(itemT_hbm.at[:, pl.ds(bbase, BW)], idx_all)

    pltpu.sync_copy(user_hbm.at[pl.ds(bbase, BW)], uidx_v)
    ucopy = pltpu.async_copy(utable_hbm.at[uidx_v], urows_v, usem)

    def fire(s, b):
        pltpu.async_copy(table_hbm.at[idx_all.at[s]], rows[b], gsem[b])

    def drain_gather(b):
        pltpu.make_async_copy(table_hbm.at[pl.ds(0, BW)], rows[b], gsem[b]).wait()

    def wait_wb(b):
        pltpu.make_async_copy(item_out.at[0, :, 0], uo[b], wsem[b]).wait()

    def transpose_tile(src, dst):
        @pl.loop(0, BW // L)
        def _k(k):
            for jj in range(L):
                pad[jj, pl.ds(0, L)] = src[k * L + jj, pl.ds(0, L)]
                pad[jj, pl.ds(L, L)] = src[k * L + jj, pl.ds(L, L)]
            for c in range(DIM):
                cvec = jnp.full((L,), c, jnp.int32)
                v = plsc.load_gather(pad, [iota, cvec])
                dst[c // 8, c % 8, pl.ds(k * L, L)] = v

    def writeback(s, b):
        pltpu.async_copy(uo[b], item_out.at[s, :, wid], wsem[b])

    for b in range(NBUF):
        fire(b, b)
    for b in range(NBUF):
        drain_gather(b)
        transpose_tile(rows[b], uo[b])
        writeback(b, b)
        fire(NBUF + b, b)

    @pl.loop(1, NBLK - 1)
    def _round(i):
        s0 = i * NBUF
        for b in range(NBUF):
            drain_gather(b)
            wait_wb(b)
            transpose_tile(rows[b], uo[b])
            writeback(s0 + b, b)
            fire(s0 + NBUF + b, b)

    for b in range(NBUF):
        drain_gather(b)
        wait_wb(b)
        transpose_tile(rows[b], uo[b])
        writeback(SEQ - NBUF + b, b)

    ucopy.wait()
    wait_wb(0)
    transpose_tile(urows_v, uo[0])
    pltpu.async_copy(uo[0], user_out.at[:, wid], wsem[0])
    for b in range(NBUF):
        wait_wb(b)


def _mask_body(mask_ref, out_ref):
    out_ref[...] = jnp.sum(mask_ref[...], axis=0).astype(jnp.int32)


def kernel(item, nbr_mask, user_id, item_input_lookup, user_embedding_matrix):
    itemT = item.T
    tbl = jnp.pad(item_input_lookup, ((0, 0), (0, 128 - DIM)))
    utbl = jnp.pad(user_embedding_matrix, ((0, 0), (0, 128 - DIM)))
    o5, uo4 = _sc_gather(itemT, user_id, tbl, utbl)
    item_emb = jnp.transpose(o5, (2, 4, 0, 1, 3)).reshape(B, SEQ, DIM)
    user_embedding = jnp.transpose(uo4, (1, 3, 0, 2)).reshape(B, DIM)
    mask_length = pl.pallas_call(
        _mask_body,
        out_shape=jax.ShapeDtypeStruct((B,), jnp.int32),
    )(nbr_mask.T)
    return item_emb, user_embedding, mask_length


# speedup vs baseline: 1.5877x; 1.2613x over previous
import functools

import jax
import jax.numpy as jnp
from jax import lax
from jax.experimental import pallas as pl
from jax.experimental.pallas import tpu as pltpu
from jax.experimental.pallas import tpu_sc as plsc

N_MID = 1000000
USER_COUNT = 100000
DIM = 32
B = 4096
SEQ = 200

NC = 2
NS = 16
NW = NC * NS

BW = B // NW
NBUF = 4
NBLK = SEQ // NBUF
L = 16

_mesh = plsc.VectorSubcoreMesh(
    core_axis_name="c", subcore_axis_name="s", num_cores=NC, num_subcores=NS
)


@functools.partial(
    pl.kernel,
    out_type=(
        jax.ShapeDtypeStruct((SEQ, DIM // 8, B // BW, 8, BW), jnp.float32),
        jax.ShapeDtypeStruct((DIM // 8, B // BW, 8, BW), jnp.float32),
    ),
    mesh=_mesh,
    scratch_types=(
        [pltpu.VMEM((SEQ, BW), jnp.int32)]
        + [pltpu.VMEM((BW, 128), jnp.float32)] * NBUF
        + [pltpu.VMEM((L, DIM + 1), jnp.float32)]
        + [pltpu.VMEM((DIM // 8, 8, BW), jnp.float32)] * NBUF
        + [
            pltpu.VMEM((BW,), jnp.int32),
            pltpu.VMEM((BW, 128), jnp.float32),
        ]
        + [pltpu.SemaphoreType.DMA] * (2 * NBUF + 1)
    ),
    compiler_params=pltpu.CompilerParams(
        use_tc_tiling_on_sc=False, needs_layout_passes=False
    ),
)
def _sc_gather(
    itemT_hbm, user_hbm, table_hbm, utable_hbm,
    item_out, user_out,
    idx_all, *ring,
):
    rows = ring[:NBUF]
    pad = ring[NBUF]
    uo = ring[NBUF + 1:2 * NBUF + 1]
    uidx_v, urows_v = ring[2 * NBUF + 1:2 * NBUF + 3]
    gsem = ring[2 * NBUF + 3:3 * NBUF + 3]
    wsem = ring[3 * NBUF + 3:4 * NBUF + 3]
    usem = ring[4 * NBUF + 3]
    wid = lax.axis_index("s") * NC + lax.axis_index("c")
    bbase = wid * BW
    iota = lax.iota(jnp.int32, L)

    pltpu.sync_copy(itemT_hbm.at[:, pl.ds(bbase, BW)], idx_all)

    pltpu.sync_copy(user_hbm.at[pl.ds(bbase, BW)], uidx_v)
    ucopy = pltpu.async_copy(utable_hbm.at[uidx_v], urows_v, usem)

    def fire(s, b):
        pltpu.async_copy(table_hbm.at[idx_all.at[s]], rows[b], gsem[b])

    def drain_gather(b):
        pltpu.make_async_copy(table_hbm.at[pl.ds(0, BW)], rows[b], gsem[b]).wait()

    def wait_wb(b):
        pltpu.make_async_copy(item_out.at[0, :, 0], uo[b], wsem[b]).wait()

    def transpose_tile(src, dst):
        @pl.loop(0, BW // L)
        def _k(k):
            for jj in range(L):
                pad[jj, pl.ds(0, L)] = src[k * L + jj, pl.ds(0, L)]
                pad[jj, pl.ds(L, L)] = src[k * L + jj, pl.ds(L, L)]
            for g in range(DIM // 8):
                vs = [
                    plsc.load_gather(
                        pad, [iota, jnp.full((L,), 8 * g + ci, jnp.int32)]
                    )
                    for ci in range(8)
                ]
                for ci in range(8):
                    dst[g, ci, pl.ds(k * L, L)] = vs[ci]

    def writeback(s, b):
        pltpu.async_copy(uo[b], item_out.at[s, :, wid], wsem[b])

    for b in range(NBUF):
        fire(b, b)
    for b in range(NBUF):
        drain_gather(b)
        transpose_tile(rows[b], uo[b])
        writeback(b, b)
        fire(NBUF + b, b)

    @pl.loop(1, NBLK - 1)
    def _round(i):
        s0 = i * NBUF
        for b in range(NBUF):
            drain_gather(b)
            wait_wb(b)
            transpose_tile(rows[b], uo[b])
            writeback(s0 + b, b)
            fire(s0 + NBUF + b, b)

    for b in range(NBUF):
        drain_gather(b)
        wait_wb(b)
        transpose_tile(rows[b], uo[b])
        writeback(SEQ - NBUF + b, b)

    ucopy.wait()
    wait_wb(0)
    transpose_tile(urows_v, uo[0])
    pltpu.async_copy(uo[0], user_out.at[:, wid], wsem[0])
    for b in range(NBUF):
        wait_wb(b)


def _mask_body(mask_ref, out_ref):
    out_ref[...] = jnp.sum(mask_ref[...], axis=0).astype(jnp.int32)


def kernel(item, nbr_mask, user_id, item_input_lookup, user_embedding_matrix):
    itemT = item.T
    tbl = jnp.pad(item_input_lookup, ((0, 0), (0, 128 - DIM)))
    utbl = jnp.pad(user_embedding_matrix, ((0, 0), (0, 128 - DIM)))
    o5, uo4 = _sc_gather(itemT, user_id, tbl, utbl)
    item_emb = jnp.transpose(o5, (2, 4, 0, 1, 3)).reshape(B, SEQ, DIM)
    user_embedding = jnp.transpose(uo4, (1, 3, 0, 2)).reshape(B, DIM)
    mask_length = pl.pallas_call(
        _mask_body,
        out_shape=jax.ShapeDtypeStruct((B,), jnp.int32),
    )(nbr_mask.T)
    return item_emb, user_embedding, mask_length


# speedup vs baseline: 1.7362x; 1.0936x over previous
import functools

import jax
import jax.numpy as jnp
from jax import lax
from jax.experimental import pallas as pl
from jax.experimental.pallas import tpu as pltpu
from jax.experimental.pallas import tpu_sc as plsc

N_MID = 1000000
USER_COUNT = 100000
DIM = 32
B = 4096
SEQ = 200

NC = 2
NS = 16
NW = NC * NS

BW = B // NW
NBUF = 4
NBLK = SEQ // NBUF
L = 16

_mesh = plsc.VectorSubcoreMesh(
    core_axis_name="c", subcore_axis_name="s", num_cores=NC, num_subcores=NS
)


@functools.partial(
    pl.kernel,
    out_type=(
        jax.ShapeDtypeStruct((SEQ, DIM // 8, B // BW, 8, BW), jnp.float32),
        jax.ShapeDtypeStruct((DIM // 8, B // BW, 8, BW), jnp.float32),
    ),
    mesh=_mesh,
    scratch_types=(
        [pltpu.VMEM((SEQ, BW), jnp.int32)]
        + [pltpu.VMEM((BW, 128), jnp.float32)] * NBUF
        + [pltpu.VMEM((L, DIM + 1), jnp.float32)]
        + [pltpu.VMEM((DIM // 8, 8, BW), jnp.float32)] * NBUF
        + [
            pltpu.VMEM((BW,), jnp.int32),
            pltpu.VMEM((BW, 128), jnp.float32),
        ]
        + [pltpu.SemaphoreType.DMA] * (2 * NBUF + 1)
    ),
    compiler_params=pltpu.CompilerParams(
        use_tc_tiling_on_sc=False, needs_layout_passes=False
    ),
)
def _sc_gather(
    itemT_hbm, user_hbm, table_hbm, utable_hbm,
    item_out, user_out,
    idx_all, *ring,
):
    rows = ring[:NBUF]
    pad = ring[NBUF]
    uo = ring[NBUF + 1:2 * NBUF + 1]
    uidx_v, urows_v = ring[2 * NBUF + 1:2 * NBUF + 3]
    gsem = ring[2 * NBUF + 3:3 * NBUF + 3]
    wsem = ring[3 * NBUF + 3:4 * NBUF + 3]
    usem = ring[4 * NBUF + 3]
    wid = lax.axis_index("s") * NC + lax.axis_index("c")
    bbase = wid * BW
    iota = lax.iota(jnp.int32, L)

    pltpu.sync_copy(itemT_hbm.at[:, pl.ds(bbase, BW)], idx_all)

    pltpu.sync_copy(user_hbm.at[pl.ds(bbase, BW)], uidx_v)
    ucopy = pltpu.async_copy(utable_hbm.at[uidx_v], urows_v, usem)

    def fire(s, b):
        pltpu.async_copy(table_hbm.at[idx_all.at[s]], rows[b], gsem[b])

    def drain_gather(b):
        pltpu.make_async_copy(table_hbm.at[pl.ds(0, BW)], rows[b], gsem[b]).wait()

    def wait_wb(b):
        pltpu.make_async_copy(item_out.at[0, :, 0], uo[b], wsem[b]).wait()

    def transpose_tile(src, dst):
        @pl.loop(0, BW // L)
        def _k(k):
            lo = [src[k * L + jj, pl.ds(0, L)] for jj in range(L)]
            hi = [src[k * L + jj, pl.ds(L, L)] for jj in range(L)]
            for jj in range(L):
                pad[jj, pl.ds(0, L)] = lo[jj]
                pad[jj, pl.ds(L, L)] = hi[jj]
            for g in range(DIM // L):
                vs = [
                    plsc.load_gather(
                        pad, [iota, jnp.full((L,), L * g + ci, jnp.int32)]
                    )
                    for ci in range(L)
                ]
                for ci in range(L):
                    c = L * g + ci
                    dst[c // 8, c % 8, pl.ds(k * L, L)] = vs[ci]

    def writeback(s, b):
        pltpu.async_copy(uo[b], item_out.at[s, :, wid], wsem[b])

    for b in range(NBUF):
        fire(b, b)
    for b in range(NBUF):
        drain_gather(b)
        transpose_tile(rows[b], uo[b])
        writeback(b, b)
        fire(NBUF + b, b)

    @pl.loop(1, NBLK - 1)
    def _round(i):
        s0 = i * NBUF
        for b in range(NBUF):
            drain_gather(b)
            wait_wb(b)
            transpose_tile(rows[b], uo[b])
            writeback(s0 + b, b)
            fire(s0 + NBUF + b, b)

    for b in range(NBUF):
        drain_gather(b)
        wait_wb(b)
        transpose_tile(rows[b], uo[b])
        writeback(SEQ - NBUF + b, b)

    ucopy.wait()
    wait_wb(0)
    transpose_tile(urows_v, uo[0])
    pltpu.async_copy(uo[0], user_out.at[:, wid], wsem[0])
    for b in range(NBUF):
        wait_wb(b)


def _mask_body(mask_ref, out_ref):
    out_ref[...] = jnp.sum(mask_ref[...], axis=0).astype(jnp.int32)


def kernel(item, nbr_mask, user_id, item_input_lookup, user_embedding_matrix):
    itemT = item.T
    tbl = jnp.pad(item_input_lookup, ((0, 0), (0, 128 - DIM)))
    utbl = jnp.pad(user_embedding_matrix, ((0, 0), (0, 128 - DIM)))
    o5, uo4 = _sc_gather(itemT, user_id, tbl, utbl)
    item_emb = jnp.transpose(o5, (2, 4, 0, 1, 3)).reshape(B, SEQ, DIM)
    user_embedding = jnp.transpose(uo4, (1, 3, 0, 2)).reshape(B, DIM)
    mask_length = pl.pallas_call(
        _mask_body,
        out_shape=jax.ShapeDtypeStruct((B,), jnp.int32),
    )(nbr_mask.T)
    return item_emb, user_embedding, mask_length


# speedup vs baseline: 1.8807x; 1.0832x over previous
import functools

import jax
import jax.numpy as jnp
from jax import lax
from jax.experimental import pallas as pl
from jax.experimental.pallas import tpu as pltpu
from jax.experimental.pallas import tpu_sc as plsc

N_MID = 1000000
USER_COUNT = 100000
DIM = 32
B = 4096
SEQ = 200

NC = 2
NS = 16
NW = NC * NS

BW = B // NW
NBUF = 8
NBLK = SEQ // NBUF
L = 16

_mesh = plsc.VectorSubcoreMesh(
    core_axis_name="c", subcore_axis_name="s", num_cores=NC, num_subcores=NS
)


@functools.partial(
    pl.kernel,
    out_type=(
        jax.ShapeDtypeStruct((SEQ, DIM // 8, B // BW, 8, BW), jnp.float32),
        jax.ShapeDtypeStruct((DIM // 8, B // BW, 8, BW), jnp.float32),
    ),
    mesh=_mesh,
    scratch_types=(
        [pltpu.VMEM((SEQ, BW), jnp.int32)]
        + [pltpu.VMEM((BW, DIM), jnp.float32)] * NBUF
        + [pltpu.VMEM((L, DIM + 1), jnp.float32)]
        + [pltpu.VMEM((DIM // 8, 8, BW), jnp.float32)] * NBUF
        + [
            pltpu.VMEM((BW,), jnp.int32),
            pltpu.VMEM((BW, DIM), jnp.float32),
        ]
        + [pltpu.SemaphoreType.DMA] * (2 * NBUF + 1)
    ),
    compiler_params=pltpu.CompilerParams(
        use_tc_tiling_on_sc=False, needs_layout_passes=False
    ),
)
def _sc_gather(
    itemT_hbm, user_hbm, table_hbm, utable_hbm,
    item_out, user_out,
    idx_all, *ring,
):
    rows = ring[:NBUF]
    pad = ring[NBUF]
    uo = ring[NBUF + 1:2 * NBUF + 1]
    uidx_v, urows_v = ring[2 * NBUF + 1:2 * NBUF + 3]
    gsem = ring[2 * NBUF + 3:3 * NBUF + 3]
    wsem = ring[3 * NBUF + 3:4 * NBUF + 3]
    usem = ring[4 * NBUF + 3]
    wid = lax.axis_index("s") * NC + lax.axis_index("c")
    bbase = wid * BW
    iota = lax.iota(jnp.int32, L)

    pltpu.sync_copy(itemT_hbm.at[:, pl.ds(bbase, BW)], idx_all)

    pltpu.sync_copy(user_hbm.at[pl.ds(bbase, BW)], uidx_v)
    ucopy = pltpu.async_copy(utable_hbm.at[uidx_v], urows_v, usem)

    def fire(s, b):
        pltpu.async_copy(table_hbm.at[idx_all.at[s]], rows[b], gsem[b])

    def drain_gather(b):
        pltpu.make_async_copy(table_hbm.at[pl.ds(0, BW)], rows[b], gsem[b]).wait()

    def wait_wb(b):
        pltpu.make_async_copy(item_out.at[0, :, 0], uo[b], wsem[b]).wait()

    def transpose_tile(src, dst):
        @pl.loop(0, BW // L)
        def _k(k):
            lo = [src[k * L + jj, pl.ds(0, L)] for jj in range(L)]
            hi = [src[k * L + jj, pl.ds(L, L)] for jj in range(L)]
            for jj in range(L):
                pad[jj, pl.ds(0, L)] = lo[jj]
                pad[jj, pl.ds(L, L)] = hi[jj]
            for g in range(DIM // L):
                vs = [
                    plsc.load_gather(
                        pad, [iota, jnp.full((L,), L * g + ci, jnp.int32)]
                    )
                    for ci in range(L)
                ]
                for ci in range(L):
                    c = L * g + ci
                    dst[c // 8, c % 8, pl.ds(k * L, L)] = vs[ci]

    def writeback(s, b):
        pltpu.async_copy(uo[b], item_out.at[s, :, wid], wsem[b])

    for b in range(NBUF):
        fire(b, b)
    for b in range(NBUF):
        drain_gather(b)
        transpose_tile(rows[b], uo[b])
        writeback(b, b)
        fire(NBUF + b, b)

    @pl.loop(1, NBLK - 1)
    def _round(i):
        s0 = i * NBUF
        for b in range(NBUF):
            drain_gather(b)
            wait_wb(b)
            transpose_tile(rows[b], uo[b])
            writeback(s0 + b, b)
            fire(s0 + NBUF + b, b)

    for b in range(NBUF):
        drain_gather(b)
        wait_wb(b)
        transpose_tile(rows[b], uo[b])
        writeback(SEQ - NBUF + b, b)

    ucopy.wait()
    wait_wb(0)
    transpose_tile(urows_v, uo[0])
    pltpu.async_copy(uo[0], user_out.at[:, wid], wsem[0])
    for b in range(NBUF):
        wait_wb(b)


def _mask_body(mask_ref, out_ref):
    out_ref[...] = jnp.sum(mask_ref[...], axis=0).astype(jnp.int32)


def kernel(item, nbr_mask, user_id, item_input_lookup, user_embedding_matrix):
    itemT = item.T
    o5, uo4 = _sc_gather(itemT, user_id, item_input_lookup, user_embedding_matrix)
    item_emb = jnp.transpose(o5, (2, 4, 0, 1, 3)).reshape(B, SEQ, DIM)
    user_embedding = jnp.transpose(uo4, (1, 3, 0, 2)).reshape(B, DIM)
    mask_length = pl.pallas_call(
        _mask_body,
        out_shape=jax.ShapeDtypeStruct((B,), jnp.int32),
    )(nbr_mask.T)
    return item_emb, user_embedding, mask_length
